# baseline (speedup 1.0000x reference)
import functools
import math

import jax
import jax.numpy as jnp
from jax import lax
from jax.experimental import pallas as pl
from jax.experimental.pallas import tpu as pltpu
from jax.experimental.pallas import tpu_sc as plsc

_VOCAB = 1000000
_D = 64
_S = 200
_B = 4096
_XPAD = 256

_NC = 2
_NS = 16
_NW = _NC * _NS
_BPW = _B // _NW
_QROWS = 32
_LANES = 16
_DSLICES = _D // _LANES
_UA = 104
_UB = _S - _UA
_GA = _UA // _LANES
_GB = _UB // _LANES


def _emb_body(x_hbm, emb_hbm, pos_hbm, out_hbm,
              blk_v, idxa_v, idxb_v, pos_v, ina_v, inb_v, outa_v, outb_v,
              gsema, gsemb, ssema, ssemb, bsem):
    wid = lax.axis_index("s") * _NC + lax.axis_index("c")
    b0 = wid * _BPW

    pltpu.sync_copy(x_hbm.at[pl.ds(b0, _QROWS)], blk_v.at[0])
    pltpu.sync_copy(pos_hbm, pos_v)
    pltpu.async_copy(x_hbm.at[pl.ds(b0 + _QROWS, _QROWS)], blk_v.at[1], bsem)

    def prep_and_fire(r):
        qph = (r // _QROWS) & 1
        rq = r % _QROWS
        for k in range(7):
            idxa_v[pl.ds(k * 16, 16)] = lax.shift_right_logical(
                blk_v[qph, rq, pl.ds(k * 16, 16)], 1)
        for k in range(7):
            idxb_v[pl.ds(k * 16, 16)] = lax.shift_right_logical(
                blk_v[qph, rq, pl.ds(96 + k * 16, 16)], 1)
        pltpu.async_copy(emb_hbm.at[idxa_v.at[pl.ds(0, _UA)]], ina_v, gsema)
        pltpu.async_copy(emb_hbm.at[idxb_v.at[pl.ds(8, _UB)]], inb_v, gsemb)

    def wait_gather_a():
        pltpu.make_async_copy(emb_hbm.at[idxa_v.at[pl.ds(0, _UA)]], ina_v,
                              gsema).wait()

    def wait_gather_b():
        pltpu.make_async_copy(emb_hbm.at[idxb_v.at[pl.ds(8, _UB)]], inb_v,
                              gsemb).wait()

    def compute_group(in_v, out_v, qph, rq, par_base, rel, j0p, posr, n):
        par_lo = (blk_v[qph, rq, pl.ds(par_base, _LANES)] & 1) * _D
        par_hi = par_lo
        if rel:
            hi = pl.multiple_of(par_base + _LANES, _LANES)
            par_hi = (blk_v[qph, rq, pl.ds(hi, _LANES)] & 1) * _D
        for k in range(_LANES - n, _LANES):
            par_k = par_lo[rel + k] if rel + k < _LANES else par_hi[rel + k - _LANES]
            j = j0p + k
            for d in range(_DSLICES):
                c = 4 * k + d
                v = in_v[j, pl.ds(pl.multiple_of(par_k + d * _LANES, _LANES),
                                  _LANES)] * 8.0 \
                    + pos_v[posr + c // 8, pl.ds((c % 8) * _LANES, _LANES)]
                out_v[j, pl.ds(d * _LANES, _LANES)] = v

    def row_pass(r, carry):
        qph = (r // _QROWS) & 1
        rq = r % _QROWS

        @pl.when(r >= 1)
        def _():
            pltpu.make_async_copy(outa_v, out_hbm.at[b0, pl.ds(0, _UA)],
                                  ssema).wait()
        wait_gather_a()

        def group_a(g, c2):
            j0p = pl.multiple_of(g * _LANES, _LANES)
            compute_group(ina_v, outa_v, qph, rq, j0p, 0, j0p, j0p // 2, _LANES)
            return c2

        lax.fori_loop(0, _GA, group_a, 0)
        compute_group(ina_v, outa_v, qph, rq, 80, 8, 88, 44, _UA - _GA * _LANES)
        pltpu.async_copy(outa_v, out_hbm.at[b0 + r, pl.ds(0, _UA)], ssema)

        @pl.when(r >= 1)
        def _():
            pltpu.make_async_copy(outb_v, out_hbm.at[b0, pl.ds(_UA, _UB)],
                                  ssemb).wait()
        wait_gather_b()

        def group_b(g, c2):
            j0p = pl.multiple_of(g * _LANES, _LANES)
            base = pl.multiple_of(96 + g * _LANES, _LANES)
            compute_group(inb_v, outb_v, qph, rq, base, 8, j0p,
                          _UA // 2 + j0p // 2, _LANES)
            return c2

        lax.fori_loop(0, _GB, group_b, 0)
        pltpu.async_copy(outb_v, out_hbm.at[b0 + r, pl.ds(_UA, _UB)], ssemb)

        @pl.when(jnp.logical_and((r + 1) % _QROWS == 0, r + 1 < _BPW))
        def _():
            pltpu.make_async_copy(x_hbm.at[pl.ds(b0, _QROWS)],
                                  blk_v.at[0], bsem).wait()

            @pl.when(r + 1 + _QROWS < _BPW)
            def _():
                q2 = (r + 1) // _QROWS + 1
                pltpu.async_copy(x_hbm.at[pl.ds(b0 + q2 * _QROWS, _QROWS)],
                                 blk_v.at[q2 & 1], bsem)

        @pl.when(r + 1 < _BPW)
        def _():
            prep_and_fire(r + 1)

        return carry

    prep_and_fire(0)
    lax.fori_loop(0, _BPW, row_pass, 0)
    pltpu.make_async_copy(outa_v, out_hbm.at[b0, pl.ds(0, _UA)], ssema).wait()
    pltpu.make_async_copy(outb_v, out_hbm.at[b0, pl.ds(_UA, _UB)], ssemb).wait()


def kernel(x, emb_table, pos_table):
    xp = jnp.pad(x, ((0, 0), (0, _XPAD - _S)))
    emb2 = emb_table.reshape(_VOCAB // 2, 2 * _D)
    pos2 = pos_table.reshape(_S // 2, 2 * _D)

    mesh = plsc.VectorSubcoreMesh(core_axis_name="c", subcore_axis_name="s")
    run = functools.partial(
        pl.kernel,
        mesh=mesh,
        out_type=jax.ShapeDtypeStruct((_B, _S, _D), jnp.float32),
        scratch_types=[
            pltpu.VMEM((2, _QROWS, _XPAD), jnp.int32),
            pltpu.VMEM((112,), jnp.int32),
            pltpu.VMEM((112,), jnp.int32),
            pltpu.VMEM((_S // 2, 2 * _D), jnp.float32),
            pltpu.VMEM((_UA, 2 * _D), jnp.float32),
            pltpu.VMEM((_UB, 2 * _D), jnp.float32),
            pltpu.VMEM((_UA, _D), jnp.float32),
            pltpu.VMEM((_UB, _D), jnp.float32),
            pltpu.SemaphoreType.DMA,
            pltpu.SemaphoreType.DMA,
            pltpu.SemaphoreType.DMA,
            pltpu.SemaphoreType.DMA,
            pltpu.SemaphoreType.DMA,
        ],
    )(_emb_body)
    return run(xp, emb2, pos2)


# speedup vs baseline: 1.3643x; 1.3643x over previous
import functools
import math

import jax
import jax.numpy as jnp
from jax import lax
from jax.experimental import pallas as pl
from jax.experimental.pallas import tpu as pltpu
from jax.experimental.pallas import tpu_sc as plsc

_VOCAB = 1000000
_D = 64
_S = 200
_B = 4096

_NC = 2
_NS = 16
_NW = _NC * _NS
_BPW = _B // _NW
_QROWS = 32
_LANES = 16
_DSLICES = _D // _LANES
_G0 = 128
_G1 = _S - _G0


def _emb_body(x_hbm, emb_hbm, pos_hbm, out_hbm,
              blk_v, pos_v, in_v, out_v, gsem0, gsem1, ssem0, ssem1, bsem):
    wid = lax.axis_index("s") * _NC + lax.axis_index("c")
    b0 = wid * _BPW

    pltpu.sync_copy(x_hbm.at[pl.ds(b0, _QROWS)], blk_v.at[0])
    pltpu.sync_copy(pos_hbm, pos_v)
    pltpu.async_copy(x_hbm.at[pl.ds(b0 + _QROWS, _QROWS)], blk_v.at[1], bsem)

    gsems = (gsem0, gsem1)
    ssems = (ssem0, ssem1)

    def fire_gathers(r, ph):
        qph = (r // _QROWS) & 1
        rq = r % _QROWS
        pltpu.async_copy(emb_hbm.at[blk_v.at[qph, rq, pl.ds(0, _G0)]],
                         in_v.at[ph, pl.ds(0, _G0)], gsems[ph])
        pltpu.async_copy(emb_hbm.at[blk_v.at[qph, rq, pl.ds(_G0, _G1)]],
                         in_v.at[ph, pl.ds(_G0, _G1)], gsems[ph])

    def wait_gathers(ph):
        pltpu.make_async_copy(emb_hbm.at[blk_v.at[0, 0, pl.ds(0, _G0)]],
                              in_v.at[ph, pl.ds(0, _G0)], gsems[ph]).wait()
        pltpu.make_async_copy(emb_hbm.at[blk_v.at[0, 0, pl.ds(_G0, _G1)]],
                              in_v.at[ph, pl.ds(_G0, _G1)], gsems[ph]).wait()

    def row_pass(r, ph):
        wait_gathers(ph)

        @pl.when(r >= 2)
        def _():
            pltpu.make_async_copy(out_v.at[ph], out_hbm.at[b0], ssems[ph]).wait()

        def seq_step(j, carry):
            for u in range(2):
                jj = 2 * j + u
                for d in range(_DSLICES):
                    v = in_v[ph, jj, pl.ds(d * _LANES, _LANES)] * 8.0 \
                        + pos_v[jj, pl.ds(d * _LANES, _LANES)]
                    out_v[ph, jj, pl.ds(d * _LANES, _LANES)] = v
            return carry

        lax.fori_loop(0, _S // 2, seq_step, 0)
        pltpu.async_copy(out_v.at[ph], out_hbm.at[b0 + r], ssems[ph])

        @pl.when(jnp.logical_and((r + 1) % _QROWS == 0,
                                 r + 1 + _QROWS < _BPW))
        def _():
            qn = (r + 1) // _QROWS + 1
            pltpu.async_copy(x_hbm.at[pl.ds(b0 + qn * _QROWS, _QROWS)],
                             blk_v.at[qn & 1], bsem)

        @pl.when(jnp.logical_and((r + 2) % _QROWS == 0, r + 2 < _BPW))
        def _():
            pltpu.make_async_copy(x_hbm.at[pl.ds(b0, _QROWS)],
                                  blk_v.at[0], bsem).wait()

        @pl.when(r + 2 < _BPW)
        def _():
            fire_gathers(r + 2, ph)

    fire_gathers(0, 0)
    fire_gathers(1, 1)

    def loop_body(i, carry):
        row_pass(2 * i, 0)
        row_pass(2 * i + 1, 1)
        return carry

    lax.fori_loop(0, _BPW // 2, loop_body, 0)
    pltpu.make_async_copy(out_v.at[0], out_hbm.at[b0], ssem0).wait()
    pltpu.make_async_copy(out_v.at[1], out_hbm.at[b0], ssem1).wait()


def kernel(x, emb_table, pos_table):
    mesh = plsc.VectorSubcoreMesh(core_axis_name="c", subcore_axis_name="s")
    run = functools.partial(
        pl.kernel,
        mesh=mesh,
        compiler_params=pltpu.CompilerParams(use_tc_tiling_on_sc=False),
        out_type=jax.ShapeDtypeStruct((_B, _S, _D), jnp.float32),
        scratch_types=[
            pltpu.VMEM((2, _QROWS, _S), jnp.int32),
            pltpu.VMEM((_S, _D), jnp.float32),
            pltpu.VMEM((2, _S, _D), jnp.float32),
            pltpu.VMEM((2, _S, _D), jnp.float32),
            pltpu.SemaphoreType.DMA,
            pltpu.SemaphoreType.DMA,
            pltpu.SemaphoreType.DMA,
            pltpu.SemaphoreType.DMA,
            pltpu.SemaphoreType.DMA,
        ],
    )(_emb_body)
    return run(x, emb_table, pos_table)


# speedup vs baseline: 1.3686x; 1.0031x over previous
import functools
import math

import jax
import jax.numpy as jnp
from jax import lax
from jax.experimental import pallas as pl
from jax.experimental.pallas import tpu as pltpu
from jax.experimental.pallas import tpu_sc as plsc

_VOCAB = 1000000
_D = 64
_S = 200
_B = 4096

_NC = 2
_NS = 16
_NW = _NC * _NS
_BPW = _B // _NW
_QROWS = 32
_LANES = 16
_DSLICES = _D // _LANES
_G0 = 128
_G1 = _S - _G0
_OROWS = _S // 2


def _emb_body(x_hbm, emb_hbm, pos_hbm, out_hbm,
              blk_v, pos_v, in_v, out_v, gsem0, gsem1, ssem0, ssem1, bsem):
    wid = lax.axis_index("s") * _NC + lax.axis_index("c")
    b0 = wid * _BPW

    pltpu.sync_copy(x_hbm.at[pl.ds(b0, _QROWS)], blk_v.at[0])
    pltpu.sync_copy(pos_hbm, pos_v)
    pltpu.async_copy(x_hbm.at[pl.ds(b0 + _QROWS, _QROWS)], blk_v.at[1], bsem)

    gsems = (gsem0, gsem1)
    ssems = (ssem0, ssem1)

    def fire_gathers(r, ph):
        qph = (r // _QROWS) & 1
        rq = r % _QROWS
        pltpu.async_copy(emb_hbm.at[blk_v.at[qph, rq, pl.ds(0, _G0)]],
                         in_v.at[ph, pl.ds(0, _G0)], gsems[ph])
        pltpu.async_copy(emb_hbm.at[blk_v.at[qph, rq, pl.ds(_G0, _G1)]],
                         in_v.at[ph, pl.ds(_G0, _G1)], gsems[ph])

    def wait_gathers(ph):
        pltpu.make_async_copy(emb_hbm.at[blk_v.at[0, 0, pl.ds(0, _G0)]],
                              in_v.at[ph, pl.ds(0, _G0)], gsems[ph]).wait()
        pltpu.make_async_copy(emb_hbm.at[blk_v.at[0, 0, pl.ds(_G0, _G1)]],
                              in_v.at[ph, pl.ds(_G0, _G1)], gsems[ph]).wait()

    def row_pass(r, ph):
        wait_gathers(ph)

        @pl.when(r >= 2)
        def _():
            pltpu.make_async_copy(out_v.at[ph],
                                  out_hbm.at[pl.ds(b0 * _OROWS, _OROWS)],
                                  ssems[ph]).wait()

        def seq_step(t, carry):
            for u in range(2):
                jj = 2 * t + u
                for d in range(_DSLICES):
                    v = in_v[ph, jj, pl.ds(d * _LANES, _LANES)] * 8.0 \
                        + pos_v[jj, pl.ds(d * _LANES, _LANES)]
                    out_v[ph, t, pl.ds(u * _D + d * _LANES, _LANES)] = v
            return carry

        lax.fori_loop(0, _OROWS, seq_step, 0)
        pltpu.async_copy(out_v.at[ph],
                         out_hbm.at[pl.ds((b0 + r) * _OROWS, _OROWS)], ssems[ph])

        @pl.when(jnp.logical_and((r + 1) % _QROWS == 0,
                                 r + 1 + _QROWS < _BPW))
        def _():
            qn = (r + 1) // _QROWS + 1
            pltpu.async_copy(x_hbm.at[pl.ds(b0 + qn * _QROWS, _QROWS)],
                             blk_v.at[qn & 1], bsem)

        @pl.when(jnp.logical_and((r + 2) % _QROWS == 0, r + 2 < _BPW))
        def _():
            pltpu.make_async_copy(x_hbm.at[pl.ds(b0, _QROWS)],
                                  blk_v.at[0], bsem).wait()

        @pl.when(r + 2 < _BPW)
        def _():
            fire_gathers(r + 2, ph)

    fire_gathers(0, 0)
    fire_gathers(1, 1)

    def loop_body(i, carry):
        row_pass(2 * i, 0)
        row_pass(2 * i + 1, 1)
        return carry

    lax.fori_loop(0, _BPW // 2, loop_body, 0)
    pltpu.make_async_copy(out_v.at[0], out_hbm.at[pl.ds(0, _OROWS)], ssem0).wait()
    pltpu.make_async_copy(out_v.at[1], out_hbm.at[pl.ds(0, _OROWS)], ssem1).wait()


def kernel(x, emb_table, pos_table):
    emb_lin = emb_table.reshape(_VOCAB // 2, 2 * _D).reshape(_VOCAB, _D)

    mesh = plsc.VectorSubcoreMesh(core_axis_name="c", subcore_axis_name="s")
    run = functools.partial(
        pl.kernel,
        mesh=mesh,
        compiler_params=pltpu.CompilerParams(use_tc_tiling_on_sc=False),
        out_type=jax.ShapeDtypeStruct((_B * _S // 2, 2 * _D), jnp.float32),
        scratch_types=[
            pltpu.VMEM((2, _QROWS, _S), jnp.int32),
            pltpu.VMEM((_S, _D), jnp.float32),
            pltpu.VMEM((2, _S, _D), jnp.float32),
            pltpu.VMEM((2, _OROWS, 2 * _D), jnp.float32),
            pltpu.SemaphoreType.DMA,
            pltpu.SemaphoreType.DMA,
            pltpu.SemaphoreType.DMA,
            pltpu.SemaphoreType.DMA,
            pltpu.SemaphoreType.DMA,
        ],
    )(_emb_body)
    out2 = run(x, emb_lin, pos_table)
    return out2.reshape(_B, _S, _D)
